# baseline (speedup 1.0000x reference)
import functools

import jax
import jax.numpy as jnp
from jax import lax
from jax.experimental import pallas as pl
from jax.experimental.pallas import tpu as pltpu
from jax.experimental.pallas import tpu_sc as plsc

B = 16
E = 768
T = 16
HW = 196
DS = 7
TOPK = 8
EC = 128
NE = E // EC
SCALE = 1.0 / (16.0 * E * T)
CC = 128
NSUB = (E // 2) // CC


def _make_pool_matrix():
    q = jnp.arange(HW)
    h, w = q // 14, q % 14
    p = (h // 2) * DS + (w // 2)
    return (p[:, None] == jnp.arange(DS * DS)[None, :]).astype(jnp.bfloat16)


def _sc_copy_body(x4, out_a4, out_d4, buf0, buf1, semr, semw0, semw1):
    cid = lax.axis_index("c")
    sid = lax.axis_index("s")
    wid = cid * 16 + sid
    b = wid // 2
    c_base = (wid % 2) * (E // 2)

    bufs = (buf0, buf1)
    semws = (semw0, semw1)
    jobs = []
    for k in range(TOPK):
        for t_src, out4 in ((2 * k, out_a4), (2 * k + 1, out_d4)):
            for sub in range(NSUB):
                jobs.append((c_base + sub * CC, t_src, k, out4))
    for j, (c0, t_src, k, out4) in enumerate(jobs):
        buf = bufs[j % 2]
        semw = semws[j % 2]
        if j >= 2:
            pc0, pt, pk, pout4 = jobs[j - 2]
            pltpu.make_async_copy(buf, pout4.at[b, pl.ds(pc0, CC), pk, :],
                                  semw).wait()
        pltpu.async_copy(x4.at[b, pl.ds(c0, CC), t_src, :], buf, semr).wait()
        pltpu.async_copy(buf, out4.at[b, pl.ds(c0, CC), k, :], semw)
    for j in (len(jobs) - 2, len(jobs) - 1):
        c0, t_src, k, out4 = jobs[j]
        pltpu.make_async_copy(bufs[j % 2], out4.at[b, pl.ds(c0, CC), k, :],
                              semws[j % 2]).wait()


@functools.cache
def _sc_copy_call():
    return functools.partial(
        pl.kernel,
        out_type=(jax.ShapeDtypeStruct((B, E, TOPK, HW), jnp.float32),
                  jax.ShapeDtypeStruct((B, E, TOPK, HW), jnp.float32)),
        mesh=plsc.VectorSubcoreMesh(core_axis_name="c", subcore_axis_name="s"),
        compiler_params=pltpu.CompilerParams(use_tc_tiling_on_sc=False),
        scratch_types=[
            pltpu.VMEM((CC, HW), jnp.float32),
            pltpu.VMEM((CC, HW), jnp.float32),
            pltpu.SemaphoreType.DMA,
            pltpu.SemaphoreType.DMA,
            pltpu.SemaphoreType.DMA,
        ],
    )(_sc_copy_body)


def _sel_from_score(s):
    sj = jnp.broadcast_to(s[None, :], (T, T))
    si = jnp.broadcast_to(s[:, None], (T, T))
    ii = lax.broadcasted_iota(jnp.int32, (T, T), 0)
    jj = lax.broadcasted_iota(jnp.int32, (T, T), 1)
    tie = (sj == si) & (jj < ii)
    rank_a = jnp.sum(((sj > si) | tie).astype(jnp.int32), axis=1)
    rank_d = jnp.sum(((sj < si) | tie).astype(jnp.int32), axis=1)
    mem_a2 = jnp.broadcast_to((rank_a < TOPK)[None, :], (T, T))
    mem_d2 = jnp.broadcast_to((rank_d < TOPK)[None, :], (T, T))
    zero = jnp.zeros((T, T), jnp.int32)
    pos_a = jnp.sum(jnp.where((jj < ii) & mem_a2, 1, zero), axis=1)
    pos_d = jnp.sum(jnp.where((jj < ii) & mem_d2, 1, zero), axis=1)
    pos_a2 = jnp.broadcast_to(pos_a[None, :], (T, T))
    pos_d2 = jnp.broadcast_to(pos_d[None, :], (T, T))
    sel_a = jnp.sum(jnp.where(mem_a2 & (pos_a2 == ii), jj, zero), axis=1)
    sel_d = jnp.sum(jnp.where(mem_d2 & (pos_d2 == ii), jj, zero), axis=1)
    return sel_a, sel_d


def _score_body(x_ref, pt_ref, sela_ref, seld_ref, acc_ref):
    e = pl.program_id(1)
    x2 = x_ref[0].reshape(EC * T, HW).astype(jnp.bfloat16)
    z = jnp.dot(x2, pt_ref[...], preferred_element_type=jnp.float32)
    z3 = z.reshape(EC, T, DS * DS)
    w = jnp.sum(z3, axis=1)
    partial = jnp.sum(z3 * w[:, None, :], axis=(0, 2)) * SCALE

    @pl.when(e == 0)
    def _():
        t_i = lax.iota(jnp.int32, T)
        acc_ref[0, :] = partial + (1 - (t_i % 2)).astype(jnp.float32)

    @pl.when(e != 0)
    def _():
        acc_ref[0, :] = acc_ref[0, :] + partial

    @pl.when(e == NE - 1)
    def _():
        sel_a, sel_d = _sel_from_score(acc_ref[0, :])
        sela_ref[0, 0, :] = sel_a
        seld_ref[0, 0, :] = sel_d


_score_call = pl.pallas_call(
    _score_body,
    grid=(B, NE),
    in_specs=[
        pl.BlockSpec((1, EC, T, HW), lambda b, e: (b, e, 0, 0)),
        pl.BlockSpec((HW, DS * DS), lambda b, e: (0, 0)),
    ],
    out_specs=[pl.BlockSpec((1, 1, T), lambda b, e: (b, 0, 0))] * 2,
    out_shape=[jax.ShapeDtypeStruct((B, 1, T), jnp.int32)] * 2,
    scratch_shapes=[pltpu.VMEM((1, T), jnp.float32)],
)


def _correct_body(sela_sm, seld_sm, x_hbm, outa_in, outd_in,
                  outa_ref, outd_ref, sem):
    b = pl.program_id(0)
    for k in range(TOPK):
        t_a = sela_sm[b * T + k]
        t_d = seld_sm[b * T + k]

        @pl.when(t_a != 2 * k)
        def _():
            cp = pltpu.make_async_copy(
                x_hbm.at[b, :, t_a, :], outa_ref.at[b, :, k, :], sem)
            cp.start()
            cp.wait()

        @pl.when(t_d != 2 * k + 1)
        def _():
            cp = pltpu.make_async_copy(
                x_hbm.at[b, :, t_d, :], outd_ref.at[b, :, k, :], sem)
            cp.start()
            cp.wait()


_correct_call = pl.pallas_call(
    _correct_body,
    grid=(B,),
    in_specs=[
        pl.BlockSpec(memory_space=pltpu.SMEM),
        pl.BlockSpec(memory_space=pltpu.SMEM),
        pl.BlockSpec(memory_space=pltpu.HBM),
        pl.BlockSpec(memory_space=pltpu.HBM),
        pl.BlockSpec(memory_space=pltpu.HBM),
    ],
    out_specs=[pl.BlockSpec(memory_space=pltpu.HBM)] * 2,
    out_shape=[jax.ShapeDtypeStruct((B, E, TOPK, HW), jnp.float32)] * 2,
    input_output_aliases={3: 0, 4: 1},
    scratch_shapes=[pltpu.SemaphoreType.DMA],
)


def kernel(x_in):
    sel_a, sel_d = _score_call(x_in, _make_pool_matrix())
    oa0, od0 = _sc_copy_call()(x_in)
    return tuple(_correct_call(sel_a.reshape(B * T), sel_d.reshape(B * T),
                               x_in, oa0, od0))


# speedup vs baseline: 2.6003x; 2.6003x over previous
import jax
import jax.numpy as jnp
from jax import lax
from jax.experimental import pallas as pl
from jax.experimental.pallas import tpu as pltpu

B = 16
E = 768
T = 16
HW = 196
DS = 7
TOPK = 8
SCALE = 1.0 / (16.0 * E * T)
NSLOT = 3


def _make_pool_matrix():
    q = jnp.arange(HW)
    h, w = q // 14, q % 14
    p = (h // 2) * DS + (w // 2)
    return (p[:, None] == jnp.arange(DS * DS)[None, :]).astype(jnp.bfloat16)


def _in_copy(x_hbm, xbuf, insems, idx, slot):
    return pltpu.make_async_copy(x_hbm.at[idx], xbuf.at[slot], insems.at[slot])


def _fused_body(x_hbm, pt_ref, outa_ref, outd_ref, xbuf, sel_ref, insems, wsem):
    b = pl.program_id(0)
    s = b % NSLOT

    @pl.when(b == 0)
    def _():
        _in_copy(x_hbm, xbuf, insems, 0, 0).start()
        _in_copy(x_hbm, xbuf, insems, 1, 1).start()

    _in_copy(x_hbm, xbuf, insems, b, s).wait()

    x = xbuf[s]
    x2 = x.reshape(E * T, HW).astype(jnp.bfloat16)
    z = jnp.dot(x2, pt_ref[...], preferred_element_type=jnp.float32)
    z3 = z.reshape(E, T, DS * DS)
    w = jnp.sum(z3, axis=1)
    sc = jnp.sum(z3 * w[:, None, :], axis=(0, 2)) * SCALE
    t_i = lax.iota(jnp.int32, T)
    sc = sc + (1 - (t_i % 2)).astype(jnp.float32)

    sj = jnp.broadcast_to(sc[None, :], (T, T))
    si = jnp.broadcast_to(sc[:, None], (T, T))
    ii = lax.broadcasted_iota(jnp.int32, (T, T), 0)
    jj = lax.broadcasted_iota(jnp.int32, (T, T), 1)
    tie = (sj == si) & (jj < ii)
    rank_a = jnp.sum(((sj > si) | tie).astype(jnp.int32), axis=1)
    rank_d = jnp.sum(((sj < si) | tie).astype(jnp.int32), axis=1)
    mem_a2 = jnp.broadcast_to((rank_a < TOPK)[None, :], (T, T))
    mem_d2 = jnp.broadcast_to((rank_d < TOPK)[None, :], (T, T))
    zero = jnp.zeros((T, T), jnp.int32)
    pos_a = jnp.sum(jnp.where((jj < ii) & mem_a2, 1, zero), axis=1)
    pos_d = jnp.sum(jnp.where((jj < ii) & mem_d2, 1, zero), axis=1)
    pos_a2 = jnp.broadcast_to(pos_a[None, :], (T, T))
    pos_d2 = jnp.broadcast_to(pos_d[None, :], (T, T))
    sel_ref[0, :] = jnp.sum(jnp.where(mem_a2 & (pos_a2 == ii), jj, zero), axis=1)
    sel_ref[1, :] = jnp.sum(jnp.where(mem_d2 & (pos_d2 == ii), jj, zero), axis=1)

    @pl.when(b > 0)
    def _():
        for k in range(TOPK):
            pltpu.make_async_copy(
                xbuf.at[s, :, 0, :], outa_ref.at[b, :, k, :], wsem).wait()
            pltpu.make_async_copy(
                xbuf.at[s, :, 0, :], outd_ref.at[b, :, k, :], wsem).wait()

    copies = []
    for k in range(TOPK):
        t_a = sel_ref[0, k]
        t_d = sel_ref[1, k]
        copies.append(pltpu.make_async_copy(
            xbuf.at[s, :, t_a, :], outa_ref.at[b, :, k, :], wsem))
        copies.append(pltpu.make_async_copy(
            xbuf.at[s, :, t_d, :], outd_ref.at[b, :, k, :], wsem))
    for c in copies:
        c.start()

    @pl.when(b + 2 < B)
    def _():
        _in_copy(x_hbm, xbuf, insems, b + 2, (b + 2) % NSLOT).start()

    @pl.when(b == B - 1)
    def _():
        for c in copies:
            c.wait()


_fused_call = pl.pallas_call(
    _fused_body,
    grid=(B,),
    in_specs=[
        pl.BlockSpec(memory_space=pltpu.HBM),
        pl.BlockSpec((HW, DS * DS), lambda b: (0, 0)),
    ],
    out_specs=[pl.BlockSpec(memory_space=pltpu.HBM)] * 2,
    out_shape=[jax.ShapeDtypeStruct((B, E, TOPK, HW), jnp.float32)] * 2,
    scratch_shapes=[
        pltpu.VMEM((NSLOT, E, T, HW), jnp.float32),
        pltpu.VMEM((2, T), jnp.int32),
        pltpu.SemaphoreType.DMA((NSLOT,)),
        pltpu.SemaphoreType.DMA,
    ],
)


def kernel(x_in):
    return tuple(_fused_call(x_in, _make_pool_matrix()))


# speedup vs baseline: 2.6034x; 1.0012x over previous
import jax
import jax.numpy as jnp
from jax import lax
from jax.experimental import pallas as pl
from jax.experimental.pallas import tpu as pltpu

B = 16
E = 768
T = 16
HW = 196
DS = 7
TOPK = 8
SCALE = 1.0 / (16.0 * E * T)
NSLOT = 3


def _make_pool_matrix():
    q = jnp.arange(HW)
    h, w = q // 14, q % 14
    p = (h // 2) * DS + (w // 2)
    return (p[:, None] == jnp.arange(DS * DS)[None, :]).astype(jnp.bfloat16)


NSPLIT = 4
ESP = E // NSPLIT


def _in_start(x_hbm, xbuf, insems, idx, slot):
    for i in range(NSPLIT):
        pltpu.make_async_copy(x_hbm.at[idx, pl.ds(i * ESP, ESP)],
                              xbuf.at[slot, pl.ds(i * ESP, ESP)],
                              insems.at[slot]).start()


def _in_wait(x_hbm, xbuf, insems, idx, slot):
    for i in range(NSPLIT):
        pltpu.make_async_copy(x_hbm.at[idx, pl.ds(i * ESP, ESP)],
                              xbuf.at[slot, pl.ds(i * ESP, ESP)],
                              insems.at[slot]).wait()


def _fused_body(x_hbm, pt_ref, outa_ref, outd_ref, xbuf, sel_ref, insems, wsem):
    b = pl.program_id(0)
    s = b % NSLOT

    @pl.when(b == 0)
    def _():
        _in_start(x_hbm, xbuf, insems, 0, 0)
        _in_start(x_hbm, xbuf, insems, 1, 1)

    _in_wait(x_hbm, xbuf, insems, b, s)

    x = xbuf[s]
    x2 = x.reshape(E * T, HW).astype(jnp.bfloat16)
    z = jnp.dot(x2, pt_ref[...], preferred_element_type=jnp.float32)
    z3 = z.reshape(E, T, DS * DS)
    w = jnp.sum(z3, axis=1)
    sc = jnp.sum(z3 * w[:, None, :], axis=(0, 2)) * SCALE
    t_i = lax.iota(jnp.int32, T)
    sc = sc + (1 - (t_i % 2)).astype(jnp.float32)

    sj = jnp.broadcast_to(sc[None, :], (T, T))
    si = jnp.broadcast_to(sc[:, None], (T, T))
    ii = lax.broadcasted_iota(jnp.int32, (T, T), 0)
    jj = lax.broadcasted_iota(jnp.int32, (T, T), 1)
    tie = (sj == si) & (jj < ii)
    rank_a = jnp.sum(((sj > si) | tie).astype(jnp.int32), axis=1)
    rank_d = jnp.sum(((sj < si) | tie).astype(jnp.int32), axis=1)
    mem_a2 = jnp.broadcast_to((rank_a < TOPK)[None, :], (T, T))
    mem_d2 = jnp.broadcast_to((rank_d < TOPK)[None, :], (T, T))
    zero = jnp.zeros((T, T), jnp.int32)
    pos_a = jnp.sum(jnp.where((jj < ii) & mem_a2, 1, zero), axis=1)
    pos_d = jnp.sum(jnp.where((jj < ii) & mem_d2, 1, zero), axis=1)
    pos_a2 = jnp.broadcast_to(pos_a[None, :], (T, T))
    pos_d2 = jnp.broadcast_to(pos_d[None, :], (T, T))
    sel_ref[0, :] = jnp.sum(jnp.where(mem_a2 & (pos_a2 == ii), jj, zero), axis=1)
    sel_ref[1, :] = jnp.sum(jnp.where(mem_d2 & (pos_d2 == ii), jj, zero), axis=1)

    @pl.when(b > 0)
    def _():
        for k in range(TOPK):
            pltpu.make_async_copy(
                xbuf.at[s, :, 0, :], outa_ref.at[b, :, k, :], wsem).wait()
            pltpu.make_async_copy(
                xbuf.at[s, :, 0, :], outd_ref.at[b, :, k, :], wsem).wait()

    copies = []
    for k in range(TOPK):
        t_a = sel_ref[0, k]
        t_d = sel_ref[1, k]
        copies.append(pltpu.make_async_copy(
            xbuf.at[s, :, t_a, :], outa_ref.at[b, :, k, :], wsem))
        copies.append(pltpu.make_async_copy(
            xbuf.at[s, :, t_d, :], outd_ref.at[b, :, k, :], wsem))
    for c in copies:
        c.start()

    @pl.when(b + 2 < B)
    def _():
        _in_start(x_hbm, xbuf, insems, b + 2, (b + 2) % NSLOT)

    @pl.when(b == B - 1)
    def _():
        for c in copies:
            c.wait()


_fused_call = pl.pallas_call(
    _fused_body,
    grid=(B,),
    in_specs=[
        pl.BlockSpec(memory_space=pltpu.HBM),
        pl.BlockSpec((HW, DS * DS), lambda b: (0, 0)),
    ],
    out_specs=[pl.BlockSpec(memory_space=pltpu.HBM)] * 2,
    out_shape=[jax.ShapeDtypeStruct((B, E, TOPK, HW), jnp.float32)] * 2,
    scratch_shapes=[
        pltpu.VMEM((NSLOT, E, T, HW), jnp.float32),
        pltpu.VMEM((2, T), jnp.int32),
        pltpu.SemaphoreType.DMA((NSLOT,)),
        pltpu.SemaphoreType.DMA,
    ],
)


def kernel(x_in):
    return tuple(_fused_call(x_in, _make_pool_matrix()))
